# baseline (speedup 1.0000x reference)
import jax
import jax.numpy as jnp
from jax import lax
from jax.experimental import pallas as pl
from jax.experimental.pallas import tpu as pltpu
from jax.experimental.pallas import tpu_sc as plsc

_N = 10000
_NP = 10240
_E = 320000
_C = 64
_CC = 32
_K = 10
_ALPHA = 0.1
_NC = 2
_NS = 16
_NW = _NC * _NS
_RPT = _NP // _NS
_CH = 256
_NB = 4

_EPW = _E // _NW
_PADW = 240
_NCHD = (_EPW + _PADW) // _CH

_EPT = _E // _NS
_PADT = 480
_NCHT = (_EPT + _PADT) // _CH
_NJT = _NCHT // _NB
_UR = 160
_NU = _RPT // _UR


def _sc_mesh():
    return plsc.VectorSubcoreMesh(
        core_axis_name="c", subcore_axis_name="s",
        num_cores=_NC, num_subcores=_NS)


def _deg_body(d_hbm, zd_hbm, ones_hbm, out_hbm, didx_a, ones_v, deg_sh,
              semi, sems):
    c = lax.axis_index("c")
    t = lax.axis_index("s")
    ci = pltpu.async_copy(d_hbm.at[c * _NS + t], didx_a, semi)
    co = pltpu.async_copy(ones_hbm, ones_v, semi)
    pltpu.sync_copy(zd_hbm.at[pl.ds(t * _RPT, _RPT)],
                    deg_sh.at[pl.ds(t * _RPT, _RPT)])
    ci.wait()
    co.wait()
    plsc.subcore_barrier()

    def body(i, carry):
        pltpu.async_copy(ones_v, deg_sh.at[didx_a.at[i, 0]], sems, add=True)
        return carry

    lax.fori_loop(0, _NCHD, body, 0)

    def drain(i, carry):
        pltpu.make_async_copy(ones_v, deg_sh.at[didx_a.at[0, 0]], sems).wait()
        return carry

    lax.fori_loop(0, _NCHD, drain, 0)
    plsc.subcore_barrier()
    pltpu.sync_copy(deg_sh.at[pl.ds(t * _RPT, _RPT)],
                    out_hbm.at[c, pl.ds(t * _RPT, _RPT)])


def _deg_call(d2, zd, ones):
    f = pl.kernel(
        _deg_body,
        out_type=jax.ShapeDtypeStruct((_NC, _NP), jnp.float32),
        mesh=_sc_mesh(),
        scratch_types=[
            pltpu.VMEM((_NCHD, 1, _CH), jnp.int32),
            pltpu.VMEM((_CH,), jnp.float32),
            pltpu.VMEM_SHARED((_NP,), jnp.float32),
            pltpu.SemaphoreType.DMA,
            pltpu.SemaphoreType.DMA,
        ],
    )
    return f(d2, zd, ones)


def _mega_body(g0_hbm, r1_hbm, rh0_hbm, c1e_hbm, cle_hbm, s_hbm, d_hbm,
               zc_hbm, gw_hbm, ho_hbm,
               sidx_a, didx_a, rb0, rb1, rb2, rb3, c1t, gbuf, abuf, rbuf,
               acc_sh, semi, sg0, sg1, sg2, sg3, ss0, ss1, ss2, ss3,
               su0, su1, su2, su3):
    c = lax.axis_index("c")
    t = lax.axis_index("s")
    rows = (rb0, rb1, rb2, rb3)
    semg = (sg0, sg1, sg2, sg3)
    sems = (ss0, ss1, ss2, ss3)
    rbase = t * _RPT
    cols = pl.ds(c * _CC, _CC)
    ci = pltpu.async_copy(s_hbm.at[t], sidx_a, semi)
    ci2 = pltpu.async_copy(d_hbm.at[t], didx_a, semi)
    pltpu.sync_copy(zc_hbm.at[pl.ds(rbase, _RPT)],
                    acc_sh.at[pl.ds(rbase, _RPT)])
    pltpu.sync_copy(c1e_hbm.at[pl.ds(rbase, _RPT)], c1t)
    for u in range(_NU):
        rr = rbase + u * _UR
        pltpu.sync_copy(g0_hbm.at[pl.ds(rr, _UR), cols], gbuf)
        pltpu.sync_copy(gbuf, gw_hbm.at[c, pl.ds(rr, _UR)])
    ci.wait()
    ci2.wait()
    plsc.subcore_barrier()

    def hop(k, carry):
        def body(j, carry2):
            base = j * _NB
            for b in range(_NB):
                @pl.when(j > 0)
                def _():
                    pltpu.make_async_copy(
                        rows[b], acc_sh.at[didx_a.at[base + b, 0]],
                        sems[b]).wait()
                pltpu.async_copy(
                    gw_hbm.at[c].at[sidx_a.at[base + b, 0]], rows[b], semg[b])
            for b in range(_NB):
                pltpu.make_async_copy(
                    gw_hbm.at[c].at[sidx_a.at[base + b, 0]], rows[b],
                    semg[b]).wait()
                pltpu.async_copy(
                    rows[b], acc_sh.at[didx_a.at[base + b, 0]], sems[b],
                    add=True)
            return carry2

        lax.fori_loop(0, _NJT, body, 0)
        for b in range(_NB):
            pltpu.make_async_copy(
                rows[b], acc_sh.at[didx_a.at[0, 0]], sems[b]).wait()
        plsc.subcore_barrier()

        @pl.when(k == _K - 1)
        def _():
            pltpu.sync_copy(cle_hbm.at[pl.ds(rbase, _RPT)], c1t)

        def upd(u, carry2):
            rr = rbase + u * _UR
            cg = pltpu.async_copy(gw_hbm.at[c, pl.ds(rr, _UR)], gbuf, su0)
            ca = pltpu.async_copy(acc_sh.at[pl.ds(rr, _UR)], abuf, su1)

            @pl.when(k < _K - 1)
            def _():
                pltpu.async_copy(r1_hbm.at[pl.ds(rr, _UR), cols], rbuf, su2)

            @pl.when(k == _K - 1)
            def _():
                pltpu.async_copy(rh0_hbm.at[pl.ds(rr, _UR), cols], rbuf, su2)

            cg.wait()
            ca.wait()
            cz = pltpu.async_copy(zc_hbm.at[pl.ds(rr, _UR)],
                                  acc_sh.at[pl.ds(rr, _UR)], su3)
            pltpu.make_async_copy(
                r1_hbm.at[pl.ds(rr, _UR), cols], rbuf, su2).wait()

            def vrow(r, carry3):
                for v2 in range(_CC // 16):
                    sl = pl.ds(v2 * 16, 16)
                    cv = c1t[u * _UR + r, sl]
                    gbuf[r, sl] = (cv * (gbuf[r, sl] + abuf[r, sl])
                                   + _ALPHA * rbuf[r, sl])
                return carry3

            lax.fori_loop(0, _UR, vrow, 0)

            @pl.when(k < _K - 1)
            def _():
                pltpu.sync_copy(gbuf, gw_hbm.at[c, pl.ds(rr, _UR)])

            @pl.when(k == _K - 1)
            def _():
                pltpu.sync_copy(gbuf, ho_hbm.at[pl.ds(rr, _UR), cols])

            cz.wait()
            return carry2

        lax.fori_loop(0, _NU, upd, 0)
        plsc.subcore_barrier()
        return carry

    lax.fori_loop(0, _K, hop, 0)


def _mega_call(g0f, h0, c1e, cle, s3, d3, zc):
    f = pl.kernel(
        _mega_body,
        out_type=[jax.ShapeDtypeStruct((_NC, _NP, _CC), jnp.float32),
                  jax.ShapeDtypeStruct((_NP, _C), jnp.float32)],
        mesh=_sc_mesh(),
        compiler_params=pltpu.CompilerParams(use_tc_tiling_on_sc=False),
        scratch_types=[
            pltpu.VMEM((_NCHT, 1, _CH), jnp.int32),
            pltpu.VMEM((_NCHT, 1, _CH), jnp.int32),
            pltpu.VMEM((_CH, _CC), jnp.float32),
            pltpu.VMEM((_CH, _CC), jnp.float32),
            pltpu.VMEM((_CH, _CC), jnp.float32),
            pltpu.VMEM((_CH, _CC), jnp.float32),
            pltpu.VMEM((_RPT, _CC), jnp.float32),
            pltpu.VMEM((_UR, _CC), jnp.float32),
            pltpu.VMEM((_UR, _CC), jnp.float32),
            pltpu.VMEM((_UR, _CC), jnp.float32),
            pltpu.VMEM_SHARED((_NP, _CC), jnp.float32),
        ] + [pltpu.SemaphoreType.DMA] * 13,
    )
    return f(g0f, g0f, h0, c1e, cle, s3, d3, zc)[1]


def _mlp_body(x_ref, w1_ref, b1_ref, w2_ref, b2_ref, h0_ref):
    x = x_ref[...]
    h = jnp.dot(x, w1_ref[...].T, preferred_element_type=jnp.float32)
    h = jnp.maximum(h + b1_ref[...], 0.0)
    h0 = jnp.dot(h, w2_ref[...].T, preferred_element_type=jnp.float32)
    h0_ref[...] = h0 + b2_ref[...]


def _pre_body(h0_ref, degp_ref, g0f_ref, c1e_ref, cle_ref):
    h0 = h0_ref[...]
    deg = 1.0 + degp_ref[:, 0:1] + degp_ref[:, 1:2]
    dinv = lax.rsqrt(deg)
    g0f_ref[...] = h0 * dinv
    ones_row = jnp.ones((1, _CC), jnp.float32)
    c1e_ref[...] = ((1.0 - _ALPHA) * dinv * dinv) * ones_row
    cle_ref[...] = ((1.0 - _ALPHA) * dinv) * ones_row


def kernel(x, edge_index, W1, b1, W2, b2):
    s = edge_index[0]
    d = edge_index[1]
    padw = jnp.broadcast_to(_N + jnp.arange(_PADW, dtype=jnp.int32),
                            (_NW, _PADW))
    d2 = jnp.concatenate([d.reshape(_NW, _EPW), padw],
                         axis=1).reshape(_NW, _NCHD, 1, _CH)
    padt = jnp.broadcast_to(
        _N + (jnp.arange(_PADT, dtype=jnp.int32) % _PADW), (_NS, _PADT))
    s3 = jnp.concatenate([s.reshape(_NS, _EPT), padt],
                         axis=1).reshape(_NS, _NCHT, 1, _CH)
    d3 = jnp.concatenate([d.reshape(_NS, _EPT), padt],
                         axis=1).reshape(_NS, _NCHT, 1, _CH)
    xp = jnp.pad(x, ((0, _NP - _N), (0, 0)))
    zd = jnp.zeros((_NP,), jnp.float32)
    zc = jnp.zeros((_NP, _CC), jnp.float32)
    ones = jnp.ones((_CH,), jnp.float32)

    degp = jnp.transpose(_deg_call(d2, zd, ones))

    h0 = pl.pallas_call(
        _mlp_body,
        out_shape=jax.ShapeDtypeStruct((_NP, _C), jnp.float32),
    )(xp, W1, b1.reshape(1, -1), W2, b2.reshape(1, -1))

    g0f, c1e, cle = pl.pallas_call(
        _pre_body,
        out_shape=[
            jax.ShapeDtypeStruct((_NP, _C), jnp.float32),
            jax.ShapeDtypeStruct((_NP, _CC), jnp.float32),
            jax.ShapeDtypeStruct((_NP, _CC), jnp.float32),
        ],
    )(h0, degp)

    ho = _mega_call(g0f, h0, c1e, cle, s3, d3, zc)
    return ho[:_N]


# speedup vs baseline: 1.0141x; 1.0141x over previous
import jax
import jax.numpy as jnp
from jax import lax
from jax.experimental import pallas as pl
from jax.experimental.pallas import tpu as pltpu
from jax.experimental.pallas import tpu_sc as plsc

_N = 10000
_NP = 10240
_E = 320000
_C = 64
_CC = 32
_K = 10
_ALPHA = 0.1
_NC = 2
_NS = 16
_NW = _NC * _NS
_RPT = _NP // _NS
_CH = 256
_NB = 4

_EPW = _E // _NW
_PADW = 240
_NCHD = (_EPW + _PADW) // _CH

_EPT = _E // _NS
_PADT = 480
_NCHT = (_EPT + _PADT) // _CH
_NJT = _NCHT // _NB
_UR = 160
_NU = _RPT // _UR


def _sc_mesh():
    return plsc.VectorSubcoreMesh(
        core_axis_name="c", subcore_axis_name="s",
        num_cores=_NC, num_subcores=_NS)


def _deg_body(d_hbm, zd_hbm, ones_hbm, out_hbm, didx_a, ones_v, deg_sh,
              semi, sems):
    c = lax.axis_index("c")
    t = lax.axis_index("s")
    ci = pltpu.async_copy(d_hbm.at[c * _NS + t], didx_a, semi)
    co = pltpu.async_copy(ones_hbm, ones_v, semi)
    pltpu.sync_copy(zd_hbm.at[pl.ds(t * _RPT, _RPT)],
                    deg_sh.at[pl.ds(t * _RPT, _RPT)])
    ci.wait()
    co.wait()
    plsc.subcore_barrier()

    def body(i, carry):
        pltpu.async_copy(ones_v, deg_sh.at[didx_a.at[i, 0]], sems, add=True)
        return carry

    lax.fori_loop(0, _NCHD, body, 0)

    def drain(i, carry):
        pltpu.make_async_copy(ones_v, deg_sh.at[didx_a.at[0, 0]], sems).wait()
        return carry

    lax.fori_loop(0, _NCHD, drain, 0)
    plsc.subcore_barrier()
    pltpu.sync_copy(deg_sh.at[pl.ds(t * _RPT, _RPT)],
                    out_hbm.at[c, pl.ds(t * _RPT, _RPT)])


def _deg_call(d2, zd, ones):
    f = pl.kernel(
        _deg_body,
        out_type=jax.ShapeDtypeStruct((_NC, _NP), jnp.float32),
        mesh=_sc_mesh(),
        scratch_types=[
            pltpu.VMEM((_NCHD, 1, _CH), jnp.int32),
            pltpu.VMEM((_CH,), jnp.float32),
            pltpu.VMEM_SHARED((_NP,), jnp.float32),
            pltpu.SemaphoreType.DMA,
            pltpu.SemaphoreType.DMA,
        ],
    )
    return f(d2, zd, ones)


def _mega_body(g0_hbm, r1_hbm, rh0_hbm, c1e_hbm, cle_hbm, s_hbm, d_hbm,
               zc_hbm, gw_hbm, ho_hbm,
               sidx_a, didx_a, rb0, rb1, rb2, rb3, c1t, gbuf, abuf, rbuf,
               acc_sh, semi, sg0, sg1, sg2, sg3, ss0, ss1, ss2, ss3,
               su0, su1, su2, su3):
    c = lax.axis_index("c")
    t = lax.axis_index("s")
    rows = (rb0, rb1, rb2, rb3)
    semg = (sg0, sg1, sg2, sg3)
    sems = (ss0, ss1, ss2, ss3)
    rbase = t * _RPT
    cols = pl.ds(c * _CC, _CC)
    ci = pltpu.async_copy(s_hbm.at[t], sidx_a, semi)
    ci2 = pltpu.async_copy(d_hbm.at[t], didx_a, semi)
    pltpu.sync_copy(zc_hbm.at[pl.ds(rbase, _RPT)],
                    acc_sh.at[pl.ds(rbase, _RPT)])
    pltpu.sync_copy(c1e_hbm.at[pl.ds(rbase, _RPT)], c1t)
    for u in range(_NU):
        rr = rbase + u * _UR
        pltpu.sync_copy(g0_hbm.at[pl.ds(rr, _UR), cols], gbuf)
        pltpu.sync_copy(gbuf, gw_hbm.at[c, pl.ds(rr, _UR)])
    ci.wait()
    ci2.wait()
    plsc.subcore_barrier()

    def hop(k, carry):
        def body(j, carry2):
            base = j * _NB
            for b in range(_NB):
                @pl.when(j > 0)
                def _():
                    pltpu.make_async_copy(
                        rows[b], acc_sh.at[didx_a.at[base + b, 0]],
                        sems[b]).wait()
                pltpu.async_copy(
                    gw_hbm.at[c].at[sidx_a.at[base + b, 0]], rows[b], semg[b])
            for b in range(_NB):
                pltpu.make_async_copy(
                    gw_hbm.at[c].at[sidx_a.at[base + b, 0]], rows[b],
                    semg[b]).wait()
                pltpu.async_copy(
                    rows[b], acc_sh.at[didx_a.at[base + b, 0]], sems[b],
                    add=True)
            return carry2

        lax.fori_loop(0, _NJT, body, 0)
        @pl.when(k == _K - 1)
        def _():
            pltpu.sync_copy(cle_hbm.at[pl.ds(rbase, _RPT)], c1t)
        pltpu.async_copy(gw_hbm.at[c, pl.ds(rbase, _UR)], gbuf, su0)

        @pl.when(k < _K - 1)
        def _():
            pltpu.async_copy(r1_hbm.at[pl.ds(rbase, _UR), cols], rbuf, su2)

        @pl.when(k == _K - 1)
        def _():
            pltpu.async_copy(rh0_hbm.at[pl.ds(rbase, _UR), cols], rbuf, su2)

        for b in range(_NB):
            pltpu.make_async_copy(
                rows[b], acc_sh.at[didx_a.at[0, 0]], sems[b]).wait()
        plsc.subcore_barrier()

        def upd(u, carry2):
            rr = rbase + u * _UR

            @pl.when(u > 0)
            def _():
                pltpu.async_copy(gw_hbm.at[c, pl.ds(rr, _UR)], gbuf, su0)

                @pl.when(k < _K - 1)
                def _():
                    pltpu.async_copy(
                        r1_hbm.at[pl.ds(rr, _UR), cols], rbuf, su2)

                @pl.when(k == _K - 1)
                def _():
                    pltpu.async_copy(
                        rh0_hbm.at[pl.ds(rr, _UR), cols], rbuf, su2)

            ca = pltpu.async_copy(acc_sh.at[pl.ds(rr, _UR)], abuf, su1)
            cg = pltpu.make_async_copy(gw_hbm.at[c, pl.ds(rr, _UR)], gbuf,
                                       su0)
            cg.wait()
            ca.wait()
            cz = pltpu.async_copy(zc_hbm.at[pl.ds(rr, _UR)],
                                  acc_sh.at[pl.ds(rr, _UR)], su3)
            pltpu.make_async_copy(
                r1_hbm.at[pl.ds(rr, _UR), cols], rbuf, su2).wait()

            def vrow(r, carry3):
                for v2 in range(_CC // 16):
                    sl = pl.ds(v2 * 16, 16)
                    cv = c1t[u * _UR + r, sl]
                    gbuf[r, sl] = (cv * (gbuf[r, sl] + abuf[r, sl])
                                   + _ALPHA * rbuf[r, sl])
                return carry3

            lax.fori_loop(0, _UR, vrow, 0)

            @pl.when(k < _K - 1)
            def _():
                pltpu.sync_copy(gbuf, gw_hbm.at[c, pl.ds(rr, _UR)])

            @pl.when(k == _K - 1)
            def _():
                pltpu.sync_copy(gbuf, ho_hbm.at[pl.ds(rr, _UR), cols])

            cz.wait()
            return carry2

        lax.fori_loop(0, _NU, upd, 0)
        plsc.subcore_barrier()
        return carry

    lax.fori_loop(0, _K, hop, 0)


def _mega_call(g0f, h0, c1e, cle, s3, d3, zc):
    f = pl.kernel(
        _mega_body,
        out_type=[jax.ShapeDtypeStruct((_NC, _NP, _CC), jnp.float32),
                  jax.ShapeDtypeStruct((_NP, _C), jnp.float32)],
        mesh=_sc_mesh(),
        compiler_params=pltpu.CompilerParams(use_tc_tiling_on_sc=False),
        scratch_types=[
            pltpu.VMEM((_NCHT, 1, _CH), jnp.int32),
            pltpu.VMEM((_NCHT, 1, _CH), jnp.int32),
            pltpu.VMEM((_CH, _CC), jnp.float32),
            pltpu.VMEM((_CH, _CC), jnp.float32),
            pltpu.VMEM((_CH, _CC), jnp.float32),
            pltpu.VMEM((_CH, _CC), jnp.float32),
            pltpu.VMEM((_RPT, _CC), jnp.float32),
            pltpu.VMEM((_UR, _CC), jnp.float32),
            pltpu.VMEM((_UR, _CC), jnp.float32),
            pltpu.VMEM((_UR, _CC), jnp.float32),
            pltpu.VMEM_SHARED((_NP, _CC), jnp.float32),
        ] + [pltpu.SemaphoreType.DMA] * 13,
    )
    return f(g0f, g0f, h0, c1e, cle, s3, d3, zc)[1]


def _mlp_body(x_ref, w1_ref, b1_ref, w2_ref, b2_ref, degp_ref,
              h0_ref, g0f_ref, c1e_ref, cle_ref):
    x = x_ref[...]
    h = jnp.dot(x, w1_ref[...].T, preferred_element_type=jnp.float32)
    h = jnp.maximum(h + b1_ref[...], 0.0)
    h0 = jnp.dot(h, w2_ref[...].T, preferred_element_type=jnp.float32)
    h0 = h0 + b2_ref[...]
    deg = 1.0 + degp_ref[:, 0:1] + degp_ref[:, 1:2]
    dinv = lax.rsqrt(deg)
    h0_ref[...] = h0
    g0f_ref[...] = h0 * dinv
    ones_row = jnp.ones((1, _CC), jnp.float32)
    c1e_ref[...] = ((1.0 - _ALPHA) * dinv * dinv) * ones_row
    cle_ref[...] = ((1.0 - _ALPHA) * dinv) * ones_row


def kernel(x, edge_index, W1, b1, W2, b2):
    s = edge_index[0]
    d = edge_index[1]
    padw = jnp.broadcast_to(_N + jnp.arange(_PADW, dtype=jnp.int32),
                            (_NW, _PADW))
    d2 = jnp.concatenate([d.reshape(_NW, _EPW), padw],
                         axis=1).reshape(_NW, _NCHD, 1, _CH)
    padt = jnp.broadcast_to(
        _N + (jnp.arange(_PADT, dtype=jnp.int32) % _PADW), (_NS, _PADT))
    s3 = jnp.concatenate([s.reshape(_NS, _EPT), padt],
                         axis=1).reshape(_NS, _NCHT, 1, _CH)
    d3 = jnp.concatenate([d.reshape(_NS, _EPT), padt],
                         axis=1).reshape(_NS, _NCHT, 1, _CH)
    xp = jnp.pad(x, ((0, _NP - _N), (0, 0)))
    zd = jnp.zeros((_NP,), jnp.float32)
    zc = jnp.zeros((_NP, _CC), jnp.float32)
    ones = jnp.ones((_CH,), jnp.float32)

    degp = jnp.transpose(_deg_call(d2, zd, ones))

    h0, g0f, c1e, cle = pl.pallas_call(
        _mlp_body,
        out_shape=[
            jax.ShapeDtypeStruct((_NP, _C), jnp.float32),
            jax.ShapeDtypeStruct((_NP, _C), jnp.float32),
            jax.ShapeDtypeStruct((_NP, _CC), jnp.float32),
            jax.ShapeDtypeStruct((_NP, _CC), jnp.float32),
        ],
    )(xp, W1, b1.reshape(1, -1), W2, b2.reshape(1, -1), degp)

    ho = _mega_call(g0f, h0, c1e, cle, s3, d3, zc)
    return ho[:_N]
